# baseline (speedup 1.0000x reference)
import functools

import jax
import jax.numpy as jnp
import numpy as np
from jax.experimental import pallas as pl
from jax.experimental.pallas import tpu as pltpu

_S = 50
_D = 64
_N_FIXED = 10
_N_RAND = 40
_BLK = 256
_THRESH = np.uint32(838861)


def _threefry2x32(k0, k1, x0, x1):
    ks2 = k0 ^ k1 ^ np.uint32(0x1BD11BDA)
    ks = (k0, k1, ks2)
    x0 = x0 + k0
    x1 = x1 + k1
    rots_a = (13, 15, 26, 6)
    rots_b = (17, 29, 16, 24)
    for g, rots in enumerate((rots_a, rots_b, rots_a, rots_b, rots_a)):
        for r in rots:
            x0 = x0 + x1
            x1 = (x1 << np.uint32(r)) | (x1 >> np.uint32(32 - r))
            x1 = x1 ^ x0
        x0 = x0 + ks[(g + 1) % 3]
        x1 = x1 + ks[(g + 2) % 3] + np.uint32(g + 1)
    return x0, x1


def _body(idx_ref, x_ref, e_ref, o_ref):
    idv = idx_ref[0].astype(jnp.uint32)
    f0, f1 = _threefry2x32(np.uint32(0), np.uint32(12345),
                           jnp.zeros_like(idv), idv)
    jrow = jax.lax.broadcasted_iota(jnp.uint32, (_N_RAND, _BLK), 0)
    b0, b1 = _threefry2x32(jnp.broadcast_to(f0, jrow.shape),
                           jnp.broadcast_to(f1, jrow.shape),
                           jnp.zeros_like(jrow), jrow)
    bits = b0 ^ b1
    bern = ((bits >> np.uint32(9)) < _THRESH).astype(jnp.float32)
    mask = jnp.concatenate(
        [jnp.ones((_N_FIXED, _BLK), jnp.float32), bern], axis=0)
    mask_t = mask.T
    mexp = jax.lax.dot_general(
        mask_t, e_ref[...],
        dimension_numbers=(((1,), (0,)), ((), ())),
        preferred_element_type=jnp.float32)
    o_ref[...] = x_ref[...] * mexp


def kernel(X, idx):
    B, S, D = X.shape
    SD = S * D
    nb = B // _BLK
    x2 = X.reshape(B, SD)
    idx3 = idx.astype(jnp.int32).reshape(nb, 1, _BLK)
    e_mat = (jax.lax.broadcasted_iota(jnp.int32, (S, SD), 1) // D ==
             jax.lax.broadcasted_iota(jnp.int32, (S, SD), 0)
             ).astype(jnp.float32)
    out = pl.pallas_call(
        _body,
        grid=(nb,),
        in_specs=[
            pl.BlockSpec((1, 1, _BLK), lambda i: (i, 0, 0)),
            pl.BlockSpec((_BLK, SD), lambda i: (i, 0)),
            pl.BlockSpec((S, SD), lambda i: (0, 0)),
        ],
        out_specs=pl.BlockSpec((_BLK, SD), lambda i: (i, 0)),
        out_shape=jax.ShapeDtypeStruct((B, SD), X.dtype),
        compiler_params=pltpu.CompilerParams(
            dimension_semantics=("arbitrary",)),
    )(idx3, x2, e_mat)
    return out.reshape(B, S, D)
